# baseline (speedup 1.0000x reference)
import functools
import math

import numpy as np
import jax
import jax.numpy as jnp
from jax import lax
from jax.experimental import pallas as pl
from jax.experimental.pallas import tpu as pltpu
from jax.experimental.pallas import tpu_sc as plsc

_B, _F, _H, _W, _C = 4, 16, 32, 32, 1024
_L = _F * _H * _W
_KEEP_RATIO = 0.25
_NKEEP = max(1, int(_L * _KEEP_RATIO))
_NDROP = _L - _NKEEP


def _build_keep_drop():
    tokens_per_frame = _H * _W
    seq_len = _L
    keep_list, drop_list = [], []
    base = jax.random.key(42)
    for b in range(_B):
        num_keep = _NKEEP
        key = jax.random.fold_in(base, b)
        keep_ratio = num_keep / seq_len
        num_frames_keep = max(1, int(_F * math.sqrt(keep_ratio)))
        num_frames_keep = min(num_frames_keep, _F)
        k1, k2 = jax.random.split(key)
        frame_indices = jnp.sort(jax.random.permutation(k1, _F)[:num_frames_keep])
        tps = min(num_keep // num_frames_keep, tokens_per_frame)
        all_idx = []
        for i in range(num_frames_keep):
            sk = jax.random.fold_in(k2, i)
            spatial = jax.random.permutation(sk, tokens_per_frame)[:tps]
            all_idx.append(frame_indices[i] * tokens_per_frame + spatial)
        indices = jnp.sort(jnp.concatenate(all_idx))
        if int(indices.shape[0]) < num_keep:
            remaining = num_keep - int(indices.shape[0])
            m = jnp.ones(seq_len, dtype=bool).at[indices].set(False)
            available = jnp.nonzero(m, size=seq_len - int(indices.shape[0]))[0]
            k3 = jax.random.fold_in(k2, 999983)
            extra = available[jax.random.permutation(k3, int(available.shape[0]))[:remaining]]
            indices = jnp.sort(jnp.concatenate([indices, extra]))
        indices = indices[:num_keep]
        mask = jnp.zeros(seq_len, dtype=bool).at[indices].set(True)
        keep_list.append(jnp.nonzero(mask, size=num_keep)[0])
        drop_list.append(jnp.nonzero(~mask, size=seq_len - num_keep)[0])
    return jnp.stack(keep_list, axis=0), jnp.stack(drop_list, axis=0)


def _keep_drop_const():
    try:
        cpu = jax.devices("cpu")[0]
        cmesh = jax.sharding.Mesh(np.array([cpu]), ("_const",))
        with jax.set_mesh(cmesh), jax.default_device(cpu):
            kb, db = _build_keep_drop()
    except Exception:
        kb, db = jax.jit(_build_keep_drop)()
    return np.asarray(kb, np.int32), np.asarray(db, np.int32)


_KEEP_BASE, _DROP_BASE = _keep_drop_const()


_NC, _NS = 2, 16
_NW = _NC * _NS
_TOTAL = _B * _NKEEP
_PER_W = _TOTAL // _NW
_CH = 32
_NCH = _PER_W // _CH


@functools.cache
def _gather_call():
    mesh = plsc.VectorSubcoreMesh(core_axis_name="c", subcore_axis_name="s")

    @functools.partial(
        pl.kernel,
        mesh=mesh,
        out_type=jax.ShapeDtypeStruct((_TOTAL, _C), jnp.float32),
        scratch_types=[
            pltpu.VMEM((_NCH, _CH), jnp.int32),
            pltpu.VMEM((_CH, _C), jnp.float32),
            pltpu.VMEM((_CH, _C), jnp.float32),
            pltpu.SemaphoreType.DMA,
            pltpu.SemaphoreType.DMA,
            pltpu.SemaphoreType.DMA,
            pltpu.SemaphoreType.DMA,
        ],
    )
    def _gather_kernel(gidx_hbm, table_hbm, out_hbm, idx_v, rows0, rows1,
                       g0, g1, s0, s1):
        wid = lax.axis_index("s") * _NC + lax.axis_index("c")
        base = wid * _PER_W
        pltpu.sync_copy(gidx_hbm.at[wid], idx_v)
        rows = (rows0, rows1)
        gsem = (g0, g1)
        ssem = (s0, s1)
        g_desc = [None, None]
        s_desc = [None, None]
        g_desc[0] = pltpu.async_copy(table_hbm.at[idx_v.at[0]], rows[0], gsem[0])
        for c in range(_NCH):
            p = c & 1
            q = p ^ 1
            g_desc[p].wait()
            if c + 1 < _NCH:
                if c >= 1:
                    s_desc[q].wait()
                g_desc[q] = pltpu.async_copy(table_hbm.at[idx_v.at[c + 1]],
                                             rows[q], gsem[q])
            s_desc[p] = pltpu.async_copy(rows[p],
                                         out_hbm.at[pl.ds(base + c * _CH, _CH)],
                                         ssem[p])
        s_desc[0].wait()
        s_desc[1].wait()

    return _gather_kernel


_NCOPY = 8


def _copy_body(x_ref, o_ref, *sems):
    descs = [
        pltpu.make_async_copy(x_ref.at[c], o_ref.at[c], sems[c])
        for c in range(_NCOPY)
    ]
    for d in descs:
        d.start()
    for d in descs:
        d.wait()


@functools.cache
def _copy_call():
    return pl.pallas_call(
        _copy_body,
        in_specs=[pl.BlockSpec(memory_space=pltpu.MemorySpace.HBM)],
        out_specs=pl.BlockSpec(memory_space=pltpu.MemorySpace.HBM),
        out_shape=jax.ShapeDtypeStruct((_NCOPY, _B * _L // _NCOPY, _C),
                                       jnp.float32),
        scratch_shapes=[pltpu.SemaphoreType.DMA] * _NCOPY,
    )


def kernel(x, seq_lens, grid_sizes):
    B, L, C = x.shape
    delta = (seq_lens - grid_sizes[:, 0] * grid_sizes[:, 1] * grid_sizes[:, 2])[:, None]
    delta = delta.astype(jnp.int32)
    keep_idx = jnp.asarray(_KEEP_BASE) + delta
    drop_idx = jnp.asarray(_DROP_BASE) + delta
    offs = (jnp.arange(B, dtype=jnp.int32) * L)[:, None]
    gidx = (keep_idx + offs).reshape(_NW, _NCH, _CH)
    table = x.reshape(B * L, C)
    sparse = _gather_call()(gidx, table).reshape(B, _NKEEP, C)
    dense_residual = _copy_call()(
        x.reshape(_NCOPY, B * L // _NCOPY, C)).reshape(B, L, C)
    return sparse, dense_residual, keep_idx, drop_idx


# speedup vs baseline: 31.8358x; 31.8358x over previous
import functools
import math

import numpy as np
import jax
import jax.numpy as jnp
from jax import lax
from jax.experimental import pallas as pl
from jax.experimental.pallas import tpu as pltpu
from jax.experimental.pallas import tpu_sc as plsc

_B, _F, _H, _W, _C = 4, 16, 32, 32, 1024
_L = _F * _H * _W
_KEEP_RATIO = 0.25
_NKEEP = max(1, int(_L * _KEEP_RATIO))
_NDROP = _L - _NKEEP


def _build_keep_drop():
    tokens_per_frame = _H * _W
    seq_len = _L
    keep_list, drop_list = [], []
    base = jax.random.key(42)
    for b in range(_B):
        num_keep = _NKEEP
        key = jax.random.fold_in(base, b)
        keep_ratio = num_keep / seq_len
        num_frames_keep = max(1, int(_F * math.sqrt(keep_ratio)))
        num_frames_keep = min(num_frames_keep, _F)
        k1, k2 = jax.random.split(key)
        frame_indices = jnp.sort(jax.random.permutation(k1, _F)[:num_frames_keep])
        tps = min(num_keep // num_frames_keep, tokens_per_frame)
        all_idx = []
        for i in range(num_frames_keep):
            sk = jax.random.fold_in(k2, i)
            spatial = jax.random.permutation(sk, tokens_per_frame)[:tps]
            all_idx.append(frame_indices[i] * tokens_per_frame + spatial)
        indices = jnp.sort(jnp.concatenate(all_idx))
        if int(indices.shape[0]) < num_keep:
            remaining = num_keep - int(indices.shape[0])
            m = jnp.ones(seq_len, dtype=bool).at[indices].set(False)
            available = jnp.nonzero(m, size=seq_len - int(indices.shape[0]))[0]
            k3 = jax.random.fold_in(k2, 999983)
            extra = available[jax.random.permutation(k3, int(available.shape[0]))[:remaining]]
            indices = jnp.sort(jnp.concatenate([indices, extra]))
        indices = indices[:num_keep]
        mask = jnp.zeros(seq_len, dtype=bool).at[indices].set(True)
        keep_list.append(jnp.nonzero(mask, size=num_keep)[0])
        drop_list.append(jnp.nonzero(~mask, size=seq_len - num_keep)[0])
    return jnp.stack(keep_list, axis=0), jnp.stack(drop_list, axis=0)


def _keep_drop_const():
    try:
        cpu = jax.devices("cpu")[0]
        cmesh = jax.sharding.Mesh(np.array([cpu]), ("_const",))
        with jax.set_mesh(cmesh), jax.default_device(cpu):
            kb, db = _build_keep_drop()
    except Exception:
        kb, db = jax.jit(_build_keep_drop)()
    return np.asarray(kb, np.int32), np.asarray(db, np.int32)


_KEEP_BASE, _DROP_BASE = _keep_drop_const()


_NC, _NS = 2, 16
_NW = _NC * _NS
_TOTAL = _B * _NKEEP
_PER_W = _TOTAL // _NW
_CH = 32
_NCH = _PER_W // _CH


_CPER_W = _B * _L // _NW
_NCC = _CPER_W // _CH


@functools.cache
def _gather_call():
    mesh = plsc.VectorSubcoreMesh(core_axis_name="c", subcore_axis_name="s")

    @functools.partial(
        pl.kernel,
        mesh=mesh,
        out_type=(
            jax.ShapeDtypeStruct((_TOTAL, _C), jnp.float32),
            jax.ShapeDtypeStruct((_B * _L, _C), jnp.float32),
        ),
        scratch_types=[
            pltpu.VMEM((_NCH, _CH), jnp.int32),
            pltpu.VMEM((_CH, _C), jnp.float32),
            pltpu.VMEM((_CH, _C), jnp.float32),
            pltpu.SemaphoreType.DMA,
            pltpu.SemaphoreType.DMA,
            pltpu.SemaphoreType.DMA,
            pltpu.SemaphoreType.DMA,
        ],
    )
    def _gather_kernel(gidx_hbm, table_hbm, out_hbm, dense_hbm,
                       idx_v, rows0, rows1, g0, g1, s0, s1):
        wid = lax.axis_index("s") * _NC + lax.axis_index("c")
        base = wid * _PER_W
        pltpu.sync_copy(gidx_hbm.at[wid], idx_v)
        rows = (rows0, rows1)
        gsem = (g0, g1)
        ssem = (s0, s1)
        g_desc = [None, None]
        s_desc = [None, None]
        g_desc[0] = pltpu.async_copy(table_hbm.at[idx_v.at[0]], rows[0], gsem[0])
        for c in range(_NCH):
            p = c & 1
            q = p ^ 1
            g_desc[p].wait()
            if c + 1 < _NCH:
                if c >= 1:
                    s_desc[q].wait()
                g_desc[q] = pltpu.async_copy(table_hbm.at[idx_v.at[c + 1]],
                                             rows[q], gsem[q])
            s_desc[p] = pltpu.async_copy(rows[p],
                                         out_hbm.at[pl.ds(base + c * _CH, _CH)],
                                         ssem[p])
        s_desc[0].wait()
        s_desc[1].wait()
        cbase = wid * _CPER_W
        for c in range(_NCC):
            p = c & 1
            q = p ^ 1
            if c == 0:
                g_desc[0] = pltpu.async_copy(
                    table_hbm.at[pl.ds(cbase, _CH)], rows[0], gsem[0])
            g_desc[p].wait()
            if c + 1 < _NCC:
                if c >= 1:
                    s_desc[q].wait()
                g_desc[q] = pltpu.async_copy(
                    table_hbm.at[pl.ds(cbase + (c + 1) * _CH, _CH)],
                    rows[q], gsem[q])
            s_desc[p] = pltpu.async_copy(
                rows[p], dense_hbm.at[pl.ds(cbase + c * _CH, _CH)], ssem[p])
        s_desc[0].wait()
        s_desc[1].wait()

    return _gather_kernel


def kernel(x, seq_lens, grid_sizes):
    B, L, C = x.shape
    delta = (seq_lens - grid_sizes[:, 0] * grid_sizes[:, 1] * grid_sizes[:, 2])[:, None]
    delta = delta.astype(jnp.int32)
    keep_idx = jnp.asarray(_KEEP_BASE) + delta
    drop_idx = jnp.asarray(_DROP_BASE) + delta
    offs = (jnp.arange(B, dtype=jnp.int32) * L)[:, None]
    gidx = (keep_idx + offs).reshape(_NW, _NCH, _CH)
    table = x.reshape(B * L, C)
    sparse, dense = _gather_call()(gidx, table)
    return (sparse.reshape(B, _NKEEP, C), dense.reshape(B, L, C),
            keep_idx, drop_idx)


# speedup vs baseline: 35.1980x; 1.1056x over previous
import functools
import math

import numpy as np
import jax
import jax.numpy as jnp
from jax import lax
from jax.experimental import pallas as pl
from jax.experimental.pallas import tpu as pltpu
from jax.experimental.pallas import tpu_sc as plsc

_B, _F, _H, _W, _C = 4, 16, 32, 32, 1024
_L = _F * _H * _W
_KEEP_RATIO = 0.25
_NKEEP = max(1, int(_L * _KEEP_RATIO))
_NDROP = _L - _NKEEP


def _build_keep_drop():
    tokens_per_frame = _H * _W
    seq_len = _L
    keep_list, drop_list = [], []
    base = jax.random.key(42)
    for b in range(_B):
        num_keep = _NKEEP
        key = jax.random.fold_in(base, b)
        keep_ratio = num_keep / seq_len
        num_frames_keep = max(1, int(_F * math.sqrt(keep_ratio)))
        num_frames_keep = min(num_frames_keep, _F)
        k1, k2 = jax.random.split(key)
        frame_indices = jnp.sort(jax.random.permutation(k1, _F)[:num_frames_keep])
        tps = min(num_keep // num_frames_keep, tokens_per_frame)
        all_idx = []
        for i in range(num_frames_keep):
            sk = jax.random.fold_in(k2, i)
            spatial = jax.random.permutation(sk, tokens_per_frame)[:tps]
            all_idx.append(frame_indices[i] * tokens_per_frame + spatial)
        indices = jnp.sort(jnp.concatenate(all_idx))
        if int(indices.shape[0]) < num_keep:
            remaining = num_keep - int(indices.shape[0])
            m = jnp.ones(seq_len, dtype=bool).at[indices].set(False)
            available = jnp.nonzero(m, size=seq_len - int(indices.shape[0]))[0]
            k3 = jax.random.fold_in(k2, 999983)
            extra = available[jax.random.permutation(k3, int(available.shape[0]))[:remaining]]
            indices = jnp.sort(jnp.concatenate([indices, extra]))
        indices = indices[:num_keep]
        mask = jnp.zeros(seq_len, dtype=bool).at[indices].set(True)
        keep_list.append(jnp.nonzero(mask, size=num_keep)[0])
        drop_list.append(jnp.nonzero(~mask, size=seq_len - num_keep)[0])
    return jnp.stack(keep_list, axis=0), jnp.stack(drop_list, axis=0)


def _keep_drop_const():
    try:
        cpu = jax.devices("cpu")[0]
        cmesh = jax.sharding.Mesh(np.array([cpu]), ("_const",))
        with jax.set_mesh(cmesh), jax.default_device(cpu):
            kb, db = _build_keep_drop()
    except Exception:
        kb, db = jax.jit(_build_keep_drop)()
    return np.asarray(kb, np.int32), np.asarray(db, np.int32)


_KEEP_BASE, _DROP_BASE = _keep_drop_const()


_NC, _NS = 2, 16
_NW = _NC * _NS
_TOTAL = _B * _NKEEP
_PER_W = _TOTAL // _NW
_CH = 32
_NCH = _PER_W // _CH


_NBUF = 3


@functools.cache
def _gather_call():
    mesh = plsc.VectorSubcoreMesh(core_axis_name="c", subcore_axis_name="s")

    @functools.partial(
        pl.kernel,
        mesh=mesh,
        out_type=jax.ShapeDtypeStruct((_TOTAL, _C), jnp.float32),
        scratch_types=(
            [pltpu.VMEM((_NCH, _CH), jnp.int32)]
            + [pltpu.VMEM((_CH, _C), jnp.float32)] * _NBUF
            + [pltpu.SemaphoreType.DMA] * (2 * _NBUF)
        ),
    )
    def _gather_kernel(gidx_hbm, table_hbm, out_hbm, idx_v, *bufs_and_sems):
        rows = bufs_and_sems[:_NBUF]
        gsem = bufs_and_sems[_NBUF:2 * _NBUF]
        ssem = bufs_and_sems[2 * _NBUF:]
        wid = lax.axis_index("s") * _NC + lax.axis_index("c")
        base = wid * _PER_W
        pltpu.sync_copy(gidx_hbm.at[wid], idx_v)
        g_desc = [None] * _NBUF
        s_desc = [None] * _NBUF
        for j in range(min(_NBUF, _NCH)):
            g_desc[j] = pltpu.async_copy(table_hbm.at[idx_v.at[j]],
                                         rows[j], gsem[j])
        for c in range(_NCH):
            p = c % _NBUF
            g_desc[p].wait()
            s_desc[p] = pltpu.async_copy(rows[p],
                                         out_hbm.at[pl.ds(base + c * _CH, _CH)],
                                         ssem[p])
            j = c + _NBUF
            if j < _NCH:
                s_desc[p].wait()
                g_desc[p] = pltpu.async_copy(table_hbm.at[idx_v.at[j]],
                                             rows[p], gsem[p])
        for c in range(max(0, _NCH - _NBUF), _NCH):
            s_desc[c % _NBUF].wait()

    return _gather_kernel


_COPY_BLK = 2048


def _copy_body(x_ref, o_ref):
    o_ref[...] = x_ref[...]


@functools.cache
def _copy_call():
    return pl.pallas_call(
        _copy_body,
        grid=(_B, _L // _COPY_BLK),
        in_specs=[pl.BlockSpec((1, _COPY_BLK, _C), lambda b, i: (b, i, 0))],
        out_specs=pl.BlockSpec((1, _COPY_BLK, _C), lambda b, i: (b, i, 0)),
        out_shape=jax.ShapeDtypeStruct((_B, _L, _C), jnp.float32),
    )


def kernel(x, seq_lens, grid_sizes):
    B, L, C = x.shape
    delta = (seq_lens - grid_sizes[:, 0] * grid_sizes[:, 1] * grid_sizes[:, 2])[:, None]
    delta = delta.astype(jnp.int32)
    keep_idx = jnp.asarray(_KEEP_BASE) + delta
    drop_idx = jnp.asarray(_DROP_BASE) + delta
    offs = (jnp.arange(B, dtype=jnp.int32) * L)[:, None]
    gidx = (keep_idx + offs).reshape(_NW, _NCH, _CH)
    table = x.reshape(B * L, C)
    sparse = _gather_call()(gidx, table).reshape(B, _NKEEP, C)
    dense_residual = _copy_call()(x)
    return sparse, dense_residual, keep_idx, drop_idx
